# baseline (speedup 1.0000x reference)
import functools

import jax
import jax.numpy as jnp
from jax import lax
from jax.experimental import pallas as pl
from jax.experimental.pallas import tpu as pltpu
from jax.experimental.pallas import tpu_sc as plsc

N_TOKENS = 8192
D_MODEL = 2048
LANES = 16
NC, NS = 2, 16
NW = NC * NS

SC_TOKENS = 2560
TC_TOKENS = N_TOKENS - SC_TOKENS
TPW = SC_TOKENS // NW
TB = 16
NGRP = TPW // TB
TC_BLOCK = 512


def _sc_gate(x_hbm, w_hbm, b_hbm, out_hbm,
             w0_v, w1_v, v_v, b_v, xbuf0, xbuf1, mat_v, o_v, sem0, sem1):
    wid = lax.axis_index("s") * NC + lax.axis_index("c")
    base = TC_TOKENS + wid * TPW

    pltpu.sync_copy(w_hbm.at[0], w0_v)
    pltpu.sync_copy(w_hbm.at[1], w1_v)
    pltpu.sync_copy(b_hbm, b_v)

    @plsc.parallel_loop(0, D_MODEL, LANES, unroll=4)
    def _vdiff_body(off):
        v_v[pl.ds(off, LANES)] = w0_v[pl.ds(off, LANES)] - w1_v[pl.ds(off, LANES)]
    bb = b_v[...]
    db = bb[0] - bb[1]

    sems = (sem0, sem1)
    bufs = (xbuf0, xbuf1)
    lane_iota = lax.iota(jnp.int32, LANES)
    col_idx = lane_iota * (LANES + 1)

    def start_group(g, buf):
        pltpu.async_copy(
            x_hbm.at[pl.ds(base + g * TB, TB)], bufs[buf], sems[buf])

    start_group(0, 0)
    for g in range(NGRP):
        buf = g % 2
        if g + 1 < NGRP:
            start_group(g + 1, 1 - buf)
        pltpu.make_async_copy(
            x_hbm.at[pl.ds(base + g * TB, TB)], bufs[buf], sems[buf]).wait()

        xb = bufs[buf]

        def dot_body(off, accs):
            vc = v_v[pl.ds(off, LANES)]
            return tuple(
                accs[t] + xb[t, pl.ds(off, LANES)] * vc
                for t in range(TB))

        accs = plsc.parallel_loop(
            0, D_MODEL, LANES, unroll=2,
            carry=tuple(jnp.zeros((LANES,), jnp.float32) for _ in range(TB)),
        )(dot_body)

        for t in range(TB):
            mat_v[pl.ds(t * (LANES + 1), LANES)] = accs[t]
        d = jnp.zeros((LANES,), jnp.float32)
        for j in range(LANES):
            d = d + plsc.load_gather(mat_v, [col_idx + j])

        p0 = 1.0 / (1.0 + jnp.exp(-(d + db)))
        pos = (g * TB + lane_iota) * 2
        plsc.store_scatter(o_v, [pos], p0)
        plsc.store_scatter(o_v, [pos + 1], 1.0 - p0)

    pltpu.sync_copy(o_v, out_hbm.at[pl.ds(wid * TPW * 2, TPW * 2)])


def _sc_part(x, W, b16):
    mesh = plsc.VectorSubcoreMesh(core_axis_name="c", subcore_axis_name="s")
    gate = functools.partial(
        pl.kernel,
        mesh=mesh,
        compiler_params=pltpu.CompilerParams(needs_layout_passes=False),
        out_type=jax.ShapeDtypeStruct((SC_TOKENS * 2,), jnp.float32),
        scratch_types=[
            pltpu.VMEM((D_MODEL,), jnp.float32),
            pltpu.VMEM((D_MODEL,), jnp.float32),
            pltpu.VMEM((D_MODEL,), jnp.float32),
            pltpu.VMEM((LANES,), jnp.float32),
            pltpu.VMEM((TB, D_MODEL), jnp.float32),
            pltpu.VMEM((TB, D_MODEL), jnp.float32),
            pltpu.VMEM((TB * (LANES + 1),), jnp.float32),
            pltpu.VMEM((TPW * 2,), jnp.float32),
            pltpu.SemaphoreType.DMA,
            pltpu.SemaphoreType.DMA,
        ],
    )(_sc_gate)
    return gate(x, W, b16).reshape(SC_TOKENS, 2)


def _tc_block(x_ref, w_ref, b_ref, o_ref):
    xb = x_ref[...]
    w = w_ref[...]
    logits = jax.lax.dot_general(
        xb, w, (((1,), (1,)), ((), ())),
        preferred_element_type=jnp.float32)
    logits = logits + b_ref[...][None, :]
    m = jnp.max(logits, axis=1, keepdims=True)
    e = jnp.exp(logits - m)
    o_ref[...] = e / jnp.sum(e, axis=1, keepdims=True)


def _tc_part(x, W, b):
    return pl.pallas_call(
        _tc_block,
        grid=(TC_TOKENS // TC_BLOCK,),
        in_specs=[
            pl.BlockSpec((TC_BLOCK, D_MODEL), lambda i: (i, 0)),
            pl.BlockSpec((2, D_MODEL), lambda i: (0, 0)),
            pl.BlockSpec((2,), lambda i: (0,)),
        ],
        out_specs=pl.BlockSpec((TC_BLOCK, 2), lambda i: (i, 0)),
        out_shape=jax.ShapeDtypeStruct((TC_TOKENS, 2), jnp.float32),
        compiler_params=pltpu.CompilerParams(
            dimension_semantics=("arbitrary",)),
    )(x, W, b)


def kernel(x, W, b):
    b16 = jnp.pad(b, (0, LANES - 2))
    sc_out = _sc_part(x, W, b16)
    tc_out = _tc_part(x[:TC_TOKENS], W, b)
    return jnp.concatenate([tc_out, sc_out], axis=0)


# speedup vs baseline: 1.6917x; 1.6917x over previous
import functools

import jax
import jax.numpy as jnp
from jax import lax
from jax.experimental import pallas as pl
from jax.experimental.pallas import tpu as pltpu
from jax.experimental.pallas import tpu_sc as plsc

N_TOKENS = 8192
D_MODEL = 2048
LANES = 16
NC, NS = 2, 16
NW = NC * NS

SC_TOKENS = 2560
TC_TOKENS = N_TOKENS - SC_TOKENS
TPW = SC_TOKENS // NW
TB = 16
NGRP = TPW // TB
TC_BLOCK = 512


def _sc_gate(x_hbm, w_hbm, b_hbm, out_hbm,
             w0_v, w1_v, v_v, b_v, xbuf0, xbuf1, mat_v, o_v, sem0, sem1):
    wid = lax.axis_index("s") * NC + lax.axis_index("c")
    base = TC_TOKENS + wid * TPW

    pltpu.sync_copy(w_hbm.at[0], w0_v)
    pltpu.sync_copy(w_hbm.at[1], w1_v)
    pltpu.sync_copy(b_hbm, b_v)

    @plsc.parallel_loop(0, D_MODEL, LANES, unroll=4)
    def _vdiff_body(off):
        v_v[pl.ds(off, LANES)] = w0_v[pl.ds(off, LANES)] - w1_v[pl.ds(off, LANES)]
    bb = b_v[...]
    db = bb[0] - bb[1]

    sems = (sem0, sem1)
    bufs = (xbuf0, xbuf1)
    lane_iota = lax.iota(jnp.int32, LANES)
    col_idx = lane_iota * (LANES + 1)

    def start_group(g, buf):
        pltpu.async_copy(
            x_hbm.at[pl.ds(base + g * TB, TB)], bufs[buf], sems[buf])

    start_group(0, 0)
    for g in range(NGRP):
        buf = g % 2
        if g + 1 < NGRP:
            start_group(g + 1, 1 - buf)
        pltpu.make_async_copy(
            x_hbm.at[pl.ds(base + g * TB, TB)], bufs[buf], sems[buf]).wait()

        xb = bufs[buf]

        def dot_body(off, accs):
            vc = v_v[pl.ds(off, LANES)]
            return tuple(
                accs[t] + xb[t, pl.ds(off, LANES)] * vc
                for t in range(TB))

        accs = plsc.parallel_loop(
            0, D_MODEL, LANES, unroll=2,
            carry=tuple(jnp.zeros((LANES,), jnp.float32) for _ in range(TB)),
        )(dot_body)

        for t in range(TB):
            mat_v[pl.ds(t * (LANES + 1), LANES)] = accs[t]
        d = jnp.zeros((LANES,), jnp.float32)
        for j in range(LANES):
            d = d + plsc.load_gather(mat_v, [col_idx + j])

        p0 = 1.0 / (1.0 + jnp.exp(-(d + db)))
        pos = (g * TB + lane_iota) * 2
        plsc.store_scatter(o_v, [pos], p0)
        plsc.store_scatter(o_v, [pos + 1], 1.0 - p0)

    pltpu.sync_copy(o_v, out_hbm.at[pl.ds(wid * TPW * 2, TPW * 2)])


def _sc_part(x, W, b16):
    mesh = plsc.VectorSubcoreMesh(core_axis_name="c", subcore_axis_name="s")
    gate = functools.partial(
        pl.kernel,
        mesh=mesh,
        compiler_params=pltpu.CompilerParams(needs_layout_passes=False),
        out_type=jax.ShapeDtypeStruct((SC_TOKENS * 2,), jnp.float32),
        scratch_types=[
            pltpu.VMEM((D_MODEL,), jnp.float32),
            pltpu.VMEM((D_MODEL,), jnp.float32),
            pltpu.VMEM((D_MODEL,), jnp.float32),
            pltpu.VMEM((LANES,), jnp.float32),
            pltpu.VMEM((TB, D_MODEL), jnp.float32),
            pltpu.VMEM((TB, D_MODEL), jnp.float32),
            pltpu.VMEM((TB * (LANES + 1),), jnp.float32),
            pltpu.VMEM((TPW * 2,), jnp.float32),
            pltpu.SemaphoreType.DMA,
            pltpu.SemaphoreType.DMA,
        ],
    )(_sc_gate)
    return gate(x, W, b16).reshape(SC_TOKENS, 2)


def _tc_block(x_ref, w_ref, b_ref, o_ref):
    xb = x_ref[...]
    w = w_ref[...]
    logits = jax.lax.dot_general(
        xb, w, (((1,), (1,)), ((), ())),
        preferred_element_type=jnp.float32)
    logits = logits + b_ref[...][None, :]
    m = jnp.max(logits, axis=1, keepdims=True)
    e = jnp.exp(logits - m)
    o_ref[...] = e / jnp.sum(e, axis=1, keepdims=True)


def _tc_part(x, W, b):
    return pl.pallas_call(
        _tc_block,
        grid=(TC_TOKENS // TC_BLOCK,),
        in_specs=[
            pl.BlockSpec((TC_BLOCK, D_MODEL), lambda i: (i, 0)),
            pl.BlockSpec((2, D_MODEL), lambda i: (0, 0)),
            pl.BlockSpec((2,), lambda i: (0,)),
        ],
        out_specs=pl.BlockSpec((TC_BLOCK, 2), lambda i: (i, 0)),
        out_shape=jax.ShapeDtypeStruct((TC_TOKENS, 2), jnp.float32),
        compiler_params=pltpu.CompilerParams(
            dimension_semantics=("arbitrary",)),
    )(x, W, b)


def kernel(x, W, b):
    b16 = jnp.pad(b, (0, LANES - 2))
    sc_out = _sc_part(x, W, b16)
    tc_out = _tc_part(x, W, b)
    return jnp.concatenate([tc_out, sc_out], axis=0)


# speedup vs baseline: 1.6958x; 1.0024x over previous
import functools

import jax
import jax.numpy as jnp
from jax import lax
from jax.experimental import pallas as pl
from jax.experimental.pallas import tpu as pltpu
from jax.experimental.pallas import tpu_sc as plsc

N_TOKENS = 8192
D_MODEL = 2048
LANES = 16
NC, NS = 2, 16
NW = NC * NS

SC_TOKENS = 2560
TC_TOKENS = N_TOKENS - SC_TOKENS
TPW = SC_TOKENS // NW
TB = 16
NGRP = TPW // TB
TC_BLOCK = 512


def _sc_gate(x_hbm, w_hbm, b_hbm, out_hbm,
             w0_v, w1_v, v_v, b_v, xbuf0, xbuf1, xbuf2, mat_v, o_v,
             sem0, sem1, sem2):
    wid = lax.axis_index("s") * NC + lax.axis_index("c")
    base = TC_TOKENS + wid * TPW

    pltpu.sync_copy(w_hbm.at[0], w0_v)
    pltpu.sync_copy(w_hbm.at[1], w1_v)
    pltpu.sync_copy(b_hbm, b_v)

    @plsc.parallel_loop(0, D_MODEL, LANES, unroll=4)
    def _vdiff_body(off):
        v_v[pl.ds(off, LANES)] = w0_v[pl.ds(off, LANES)] - w1_v[pl.ds(off, LANES)]
    bb = b_v[...]
    db = bb[0] - bb[1]

    sems = (sem0, sem1, sem2)
    bufs = (xbuf0, xbuf1, xbuf2)
    lane_iota = lax.iota(jnp.int32, LANES)
    col_idx = lane_iota * (LANES + 1)

    def start_group(g, buf):
        pltpu.async_copy(
            x_hbm.at[pl.ds(base + g * TB, TB)], bufs[buf], sems[buf])

    start_group(0, 0)
    if NGRP > 1:
        start_group(1, 1)
    for g in range(NGRP):
        buf = g % 3
        if g + 2 < NGRP:
            start_group(g + 2, (g + 2) % 3)
        pltpu.make_async_copy(
            x_hbm.at[pl.ds(base + g * TB, TB)], bufs[buf], sems[buf]).wait()

        xb = bufs[buf]

        def dot_body(off, accs):
            vc = v_v[pl.ds(off, LANES)]
            return tuple(
                accs[t] + xb[t, pl.ds(off, LANES)] * vc
                for t in range(TB))

        accs = plsc.parallel_loop(
            0, D_MODEL, LANES, unroll=2,
            carry=tuple(jnp.zeros((LANES,), jnp.float32) for _ in range(TB)),
        )(dot_body)

        for t in range(TB):
            mat_v[pl.ds(t * (LANES + 1), LANES)] = accs[t]
        d = jnp.zeros((LANES,), jnp.float32)
        for j in range(LANES):
            d = d + plsc.load_gather(mat_v, [col_idx + j])

        p0 = 1.0 / (1.0 + jnp.exp(-(d + db)))
        pos = (g * TB + lane_iota) * 2
        plsc.store_scatter(o_v, [pos], p0)
        plsc.store_scatter(o_v, [pos + 1], 1.0 - p0)

    pltpu.sync_copy(o_v, out_hbm.at[pl.ds(wid * TPW * 2, TPW * 2)])


def _sc_part(x, W, b16):
    mesh = plsc.VectorSubcoreMesh(core_axis_name="c", subcore_axis_name="s")
    gate = functools.partial(
        pl.kernel,
        mesh=mesh,
        compiler_params=pltpu.CompilerParams(needs_layout_passes=False),
        out_type=jax.ShapeDtypeStruct((SC_TOKENS * 2,), jnp.float32),
        scratch_types=[
            pltpu.VMEM((D_MODEL,), jnp.float32),
            pltpu.VMEM((D_MODEL,), jnp.float32),
            pltpu.VMEM((D_MODEL,), jnp.float32),
            pltpu.VMEM((LANES,), jnp.float32),
            pltpu.VMEM((TB, D_MODEL), jnp.float32),
            pltpu.VMEM((TB, D_MODEL), jnp.float32),
            pltpu.VMEM((TB, D_MODEL), jnp.float32),
            pltpu.VMEM((TB * (LANES + 1),), jnp.float32),
            pltpu.VMEM((TPW * 2,), jnp.float32),
            pltpu.SemaphoreType.DMA,
            pltpu.SemaphoreType.DMA,
            pltpu.SemaphoreType.DMA,
        ],
    )(_sc_gate)
    return gate(x, W, b16).reshape(SC_TOKENS, 2)


def _tc_block(x_ref, w_ref, b_ref, o_ref):
    xb = x_ref[...]
    w = w_ref[...]
    logits = jax.lax.dot_general(
        xb, w, (((1,), (1,)), ((), ())),
        preferred_element_type=jnp.float32)
    logits = logits + b_ref[...][None, :]
    m = jnp.max(logits, axis=1, keepdims=True)
    e = jnp.exp(logits - m)
    o_ref[...] = e / jnp.sum(e, axis=1, keepdims=True)


def _tc_part(x, W, b):
    return pl.pallas_call(
        _tc_block,
        grid=(TC_TOKENS // TC_BLOCK,),
        in_specs=[
            pl.BlockSpec((TC_BLOCK, D_MODEL), lambda i: (i, 0)),
            pl.BlockSpec((2, D_MODEL), lambda i: (0, 0)),
            pl.BlockSpec((2,), lambda i: (0,)),
        ],
        out_specs=pl.BlockSpec((TC_BLOCK, 2), lambda i: (i, 0)),
        out_shape=jax.ShapeDtypeStruct((TC_TOKENS, 2), jnp.float32),
        compiler_params=pltpu.CompilerParams(
            dimension_semantics=("arbitrary",)),
    )(x, W, b)


def kernel(x, W, b):
    b16 = jnp.pad(b, (0, LANES - 2))
    sc_out = _sc_part(x, W, b16)
    tc_out = _tc_part(x, W, b)
    return jnp.concatenate([tc_out, sc_out], axis=0)


# speedup vs baseline: 1.8328x; 1.0808x over previous
import functools

import jax
import jax.numpy as jnp
from jax import lax
from jax.experimental import pallas as pl
from jax.experimental.pallas import tpu as pltpu
from jax.experimental.pallas import tpu_sc as plsc

N_TOKENS = 8192
D_MODEL = 2048
LANES = 16
NC, NS = 2, 16
NW = NC * NS

SC_TOKENS = 2048
TC_TOKENS = N_TOKENS - SC_TOKENS
TPW = SC_TOKENS // NW
TB = 16
NGRP = TPW // TB
TC_BLOCK = 1024


def _sc_gate(x_hbm, w_hbm, b_hbm, out_hbm,
             w0_v, w1_v, v_v, xbuf0, xbuf1, xbuf2, mat_v, o_v,
             sem0, sem1, sem2):
    wid = lax.axis_index("s") * NC + lax.axis_index("c")
    base = TC_TOKENS + wid * TPW

    pltpu.sync_copy(w_hbm.at[0], w0_v.at[pl.ds(0, D_MODEL)])
    pltpu.sync_copy(w_hbm.at[1], w1_v)
    pltpu.sync_copy(b_hbm, w0_v.at[pl.ds(D_MODEL, 2)])

    @plsc.parallel_loop(0, D_MODEL, LANES, unroll=4)
    def _vdiff_body(off):
        v_v[pl.ds(off, LANES)] = w0_v[pl.ds(off, LANES)] - w1_v[pl.ds(off, LANES)]
    bb = w0_v[pl.ds(D_MODEL, LANES)]
    db = bb[0] - bb[1]

    sems = (sem0, sem1, sem2)
    bufs = (xbuf0, xbuf1, xbuf2)
    lane_iota = lax.iota(jnp.int32, LANES)
    col_idx = lane_iota * (LANES + 1)

    def start_group(g, buf):
        pltpu.async_copy(
            x_hbm.at[pl.ds(base + g * TB, TB)], bufs[buf], sems[buf])

    start_group(0, 0)
    if NGRP > 1:
        start_group(1, 1)
    for g in range(NGRP):
        buf = g % 3
        if g + 2 < NGRP:
            start_group(g + 2, (g + 2) % 3)
        pltpu.make_async_copy(
            x_hbm.at[pl.ds(base + g * TB, TB)], bufs[buf], sems[buf]).wait()

        xb = bufs[buf]

        def dot_body(off, accs):
            vc = v_v[pl.ds(off, LANES)]
            return tuple(
                accs[t] + xb[t, pl.ds(off, LANES)] * vc
                for t in range(TB))

        accs = plsc.parallel_loop(
            0, D_MODEL, LANES, unroll=2,
            carry=tuple(jnp.zeros((LANES,), jnp.float32) for _ in range(TB)),
        )(dot_body)

        for t in range(TB):
            mat_v[pl.ds(t * (LANES + 1), LANES)] = accs[t]
        d = jnp.zeros((LANES,), jnp.float32)
        for j in range(LANES):
            d = d + plsc.load_gather(mat_v, [col_idx + j])

        p0 = 1.0 / (1.0 + jnp.exp(-(d + db)))
        rows = g * TB + lane_iota
        zeros_i = jnp.zeros((LANES,), jnp.int32)
        plsc.store_scatter(o_v, [rows, zeros_i], p0)
        plsc.store_scatter(o_v, [rows, zeros_i + 1], 1.0 - p0)

    pltpu.sync_copy(o_v, out_hbm.at[pl.ds(wid * TPW, TPW)])


def _sc_part(x, W, b):
    mesh = plsc.VectorSubcoreMesh(core_axis_name="c", subcore_axis_name="s")
    gate = functools.partial(
        pl.kernel,
        mesh=mesh,
        compiler_params=pltpu.CompilerParams(needs_layout_passes=False),
        out_type=jax.ShapeDtypeStruct((SC_TOKENS, 2), jnp.float32),
        scratch_types=[
            pltpu.VMEM((D_MODEL + LANES,), jnp.float32),
            pltpu.VMEM((D_MODEL,), jnp.float32),
            pltpu.VMEM((D_MODEL,), jnp.float32),
            pltpu.VMEM((TB, D_MODEL), jnp.float32),
            pltpu.VMEM((TB, D_MODEL), jnp.float32),
            pltpu.VMEM((TB, D_MODEL), jnp.float32),
            pltpu.VMEM((TB * (LANES + 1),), jnp.float32),
            pltpu.VMEM((TPW, 2), jnp.float32),
            pltpu.SemaphoreType.DMA,
            pltpu.SemaphoreType.DMA,
            pltpu.SemaphoreType.DMA,
        ],
    )(_sc_gate)
    return gate(x, W, b)


def _tc_block(x_ref, w_ref, b_ref, o_ref):
    xb = x_ref[...]
    w = w_ref[...]
    logits = jax.lax.dot_general(
        xb, w, (((1,), (1,)), ((), ())),
        preferred_element_type=jnp.float32)
    logits = logits + b_ref[...][None, :]
    m = jnp.max(logits, axis=1, keepdims=True)
    e = jnp.exp(logits - m)
    o_ref[...] = e / jnp.sum(e, axis=1, keepdims=True)


def _tc_part(x, W, b):
    return pl.pallas_call(
        _tc_block,
        grid=(TC_TOKENS // TC_BLOCK,),
        in_specs=[
            pl.BlockSpec((TC_BLOCK, D_MODEL), lambda i: (i, 0)),
            pl.BlockSpec((2, D_MODEL), lambda i: (0, 0)),
            pl.BlockSpec((2,), lambda i: (0,)),
        ],
        out_specs=pl.BlockSpec((TC_BLOCK, 2), lambda i: (i, 0)),
        out_shape=jax.ShapeDtypeStruct((TC_TOKENS, 2), jnp.float32),
        compiler_params=pltpu.CompilerParams(
            dimension_semantics=("arbitrary",)),
    )(x, W, b)


def kernel(x, W, b):
    sc_out = _sc_part(x, W, b)
    tc_out = _tc_part(x, W, b)
    return jnp.concatenate([tc_out, sc_out], axis=0)
